# baseline (speedup 1.0000x reference)
import functools

import jax
import jax.numpy as jnp
from jax import lax
from jax.experimental import pallas as pl
from jax.experimental.pallas import tpu as pltpu
from jax.experimental.pallas import tpu_sc as plsc

_BATCH = 4
_SEQ = 8192
_DIM = 1024
_NW = 32
_S_PER_W = _SEQ // _NW
_SB = 8
_NSB = _S_PER_W // _SB
_NSLOT = 3
_LANES = 16
_UNROLL = 8
_DPL = _DIM // _LANES
_DPL_SHIFT = 6


def _sc_body(x_hbm, pos_hbm, out_hbm, pos_buf, x_buf,
             in_sem0, in_sem1, in_sem2, out_sem0, out_sem1, out_sem2):
    wid = lax.axis_index("s") * 2 + lax.axis_index("c")
    base = wid * _S_PER_W
    in_sems = (in_sem0, in_sem1, in_sem2)
    out_sems = (out_sem0, out_sem1, out_sem2)

    def in_copies(ch, slot):
        pos0 = base + ch * _SB
        yield pltpu.make_async_copy(
            pos_hbm.at[pl.ds(pos0, _SB)], pos_buf.at[slot], in_sems[slot])
        for b in range(_BATCH):
            yield pltpu.make_async_copy(
                x_hbm.at[b, pl.ds(pos0, _SB)], x_buf.at[slot, b],
                in_sems[slot])

    def out_copies(ch, slot):
        pos0 = base + ch * _SB
        for b in range(_BATCH):
            yield pltpu.make_async_copy(
                x_buf.at[slot, b], out_hbm.at[b, pl.ds(pos0, _SB)],
                out_sems[slot])

    def start_in(ch, slot):
        for c in in_copies(ch, slot):
            c.start()

    def wait_in(ch, slot):
        for c in in_copies(ch, slot):
            c.wait()

    def start_out(ch, slot):
        for c in out_copies(ch, slot):
            c.start()

    def wait_out(ch, slot):
        for c in out_copies(ch, slot):
            c.wait()

    def compute(slot):
        @plsc.parallel_loop(0, _SB * _DPL, 1, unroll=_UNROLL)
        def _add(v):
            r = lax.shift_right_logical(v, _DPL_SHIFT)
            col = pl.multiple_of(
                lax.shift_left(lax.bitwise_and(v, _DPL - 1), 4), _LANES)
            sl = pl.ds(col, _LANES)
            p = pos_buf[slot, r, sl]
            for b in range(_BATCH):
                x_buf[slot, b, r, sl] = x_buf[slot, b, r, sl] + p

    start_in(0, 0)
    for i in range(_NSB):
        if i + 1 < _NSB:
            if i - 2 >= 0:
                wait_out(i - 2, (i + 1) % _NSLOT)
            start_in(i + 1, (i + 1) % _NSLOT)
        wait_in(i, i % _NSLOT)
        compute(i % _NSLOT)
        start_out(i, i % _NSLOT)
    for j in range(_NSB - 3, _NSB):
        wait_out(j, j % _NSLOT)


def kernel(x, pos_table):
    mesh = plsc.VectorSubcoreMesh(core_axis_name="c", subcore_axis_name="s")
    run = functools.partial(
        pl.kernel,
        mesh=mesh,
        out_type=jax.ShapeDtypeStruct((_BATCH, _SEQ, _DIM), jnp.float32),
        scratch_types=[
            pltpu.VMEM((_NSLOT, _SB, _DIM), jnp.float32),
            pltpu.VMEM((_NSLOT, _BATCH, _SB, _DIM), jnp.float32),
            pltpu.SemaphoreType.DMA,
            pltpu.SemaphoreType.DMA,
            pltpu.SemaphoreType.DMA,
            pltpu.SemaphoreType.DMA,
            pltpu.SemaphoreType.DMA,
            pltpu.SemaphoreType.DMA,
        ],
    )(_sc_body)
    return run(x, pos_table)


# speedup vs baseline: 1.0557x; 1.0557x over previous
import functools

import jax
import jax.numpy as jnp
from jax import lax
from jax.experimental import pallas as pl
from jax.experimental.pallas import tpu as pltpu
from jax.experimental.pallas import tpu_sc as plsc

_BATCH = 4
_SEQ = 8192
_DIM = 1024
_NW = 32
_S_PER_W = _SEQ // _NW
_SB = 4
_NSB = _S_PER_W // _SB
_LANES = 16
_UNROLL = 8
_DPL = _DIM // _LANES
_DPL_SHIFT = 6


def _sc_body(x_hbm, pos_hbm, out_hbm, pos_buf, x_buf, y_buf,
             in_sem0, in_sem1, out_sem0, out_sem1):
    wid = lax.axis_index("s") * 2 + lax.axis_index("c")
    base = wid * _S_PER_W
    in_sems = (in_sem0, in_sem1)
    out_sems = (out_sem0, out_sem1)

    def in_copies(ch, slot):
        pos0 = base + ch * _SB
        yield pltpu.make_async_copy(
            pos_hbm.at[pl.ds(pos0, _SB)], pos_buf.at[slot], in_sems[slot])
        for b in range(_BATCH):
            yield pltpu.make_async_copy(
                x_hbm.at[b, pl.ds(pos0, _SB)], x_buf.at[slot, b],
                in_sems[slot])

    def out_copies(ch, slot):
        pos0 = base + ch * _SB
        for b in range(_BATCH):
            yield pltpu.make_async_copy(
                y_buf.at[slot, b], out_hbm.at[b, pl.ds(pos0, _SB)],
                out_sems[slot])

    def start_in(ch, slot):
        for c in in_copies(ch, slot):
            c.start()

    def wait_in(ch, slot):
        for c in in_copies(ch, slot):
            c.wait()

    def start_out(ch, slot):
        for c in out_copies(ch, slot):
            c.start()

    def wait_out(ch, slot):
        for c in out_copies(ch, slot):
            c.wait()

    def compute(slot):
        @plsc.parallel_loop(0, _SB * _DPL, 1, unroll=_UNROLL)
        def _add(v):
            r = lax.shift_right_logical(v, _DPL_SHIFT)
            col = pl.multiple_of(
                lax.shift_left(lax.bitwise_and(v, _DPL - 1), 4), _LANES)
            sl = pl.ds(col, _LANES)
            p = pos_buf[slot, r, sl]
            for b in range(_BATCH):
                y_buf[slot, b, r, sl] = x_buf[slot, b, r, sl] + p

    def step(i, slot):
        @pl.when(i + 1 < _NSB)
        def _():
            start_in(i + 1, 1 - slot)

        @pl.when(i >= 2)
        def _():
            wait_out(i - 2, slot)

        wait_in(i, slot)
        compute(slot)
        start_out(i, slot)

    start_in(0, 0)

    def pair_loop(ci, carry):
        step(ci * 2, 0)
        step(ci * 2 + 1, 1)
        return carry

    lax.fori_loop(0, _NSB // 2, pair_loop, 0)
    wait_out(_NSB - 2, 0)
    wait_out(_NSB - 1, 1)


def kernel(x, pos_table):
    mesh = plsc.VectorSubcoreMesh(core_axis_name="c", subcore_axis_name="s")
    run = functools.partial(
        pl.kernel,
        mesh=mesh,
        out_type=jax.ShapeDtypeStruct((_BATCH, _SEQ, _DIM), jnp.float32),
        scratch_types=[
            pltpu.VMEM((2, _SB, _DIM), jnp.float32),
            pltpu.VMEM((2, _BATCH, _SB, _DIM), jnp.float32),
            pltpu.VMEM((2, _BATCH, _SB, _DIM), jnp.float32),
            pltpu.SemaphoreType.DMA,
            pltpu.SemaphoreType.DMA,
            pltpu.SemaphoreType.DMA,
            pltpu.SemaphoreType.DMA,
        ],
    )(_sc_body)
    return run(x, pos_table)


# speedup vs baseline: 1.0794x; 1.0224x over previous
import functools

import jax
import jax.numpy as jnp
from jax import lax
from jax.experimental import pallas as pl
from jax.experimental.pallas import tpu as pltpu
from jax.experimental.pallas import tpu_sc as plsc

_BATCH = 4
_SEQ = 8192
_DIM = 1024
_NW = 32
_S_PER_W = _SEQ // _NW
_SB = 4
_NSB = _S_PER_W // _SB
_NIN = 4
_NOUT = 2
_LANES = 16
_UNROLL = 8
_DPL = _DIM // _LANES
_DPL_SHIFT = 6


def _sc_body(x_hbm, pos_hbm, out_hbm, pos_buf, x_buf, y_buf,
             in_sem0, in_sem1, in_sem2, in_sem3, out_sem0, out_sem1):
    wid = lax.axis_index("s") * 2 + lax.axis_index("c")
    base = wid * _S_PER_W
    in_sems = (in_sem0, in_sem1, in_sem2, in_sem3)
    out_sems = (out_sem0, out_sem1)

    def in_copies(ch, slot):
        pos0 = base + ch * _SB
        yield pltpu.make_async_copy(
            pos_hbm.at[pl.ds(pos0, _SB)], pos_buf.at[slot], in_sems[slot])
        for b in range(_BATCH):
            yield pltpu.make_async_copy(
                x_hbm.at[b, pl.ds(pos0, _SB)], x_buf.at[slot, b],
                in_sems[slot])

    def out_copies(ch, slot):
        pos0 = base + ch * _SB
        for b in range(_BATCH):
            yield pltpu.make_async_copy(
                y_buf.at[slot, b], out_hbm.at[b, pl.ds(pos0, _SB)],
                out_sems[slot])

    def start_in(ch, slot):
        for c in in_copies(ch, slot):
            c.start()

    def wait_in(ch, slot):
        for c in in_copies(ch, slot):
            c.wait()

    def start_out(ch, slot):
        for c in out_copies(ch, slot):
            c.start()

    def wait_out(ch, slot):
        for c in out_copies(ch, slot):
            c.wait()

    def compute(islot, oslot):
        @plsc.parallel_loop(0, _SB * _DPL, 1, unroll=_UNROLL)
        def _add(v):
            r = lax.shift_right_logical(v, _DPL_SHIFT)
            col = pl.multiple_of(
                lax.shift_left(lax.bitwise_and(v, _DPL - 1), 4), _LANES)
            sl = pl.ds(col, _LANES)
            p = pos_buf[islot, r, sl]
            for b in range(_BATCH):
                y_buf[oslot, b, r, sl] = x_buf[islot, b, r, sl] + p

    def step(i, islot, oslot):
        @pl.when(i + 2 < _NSB)
        def _():
            start_in(i + 2, (islot + 2) % _NIN)

        @pl.when(i >= _NOUT)
        def _():
            wait_out(i - _NOUT, oslot)

        wait_in(i, islot)
        compute(islot, oslot)
        start_out(i, oslot)

    start_in(0, 0)
    start_in(1, 1)

    def quad_loop(ci, carry):
        for u in range(_NIN):
            step(ci * _NIN + u, u, u % _NOUT)
        return carry

    lax.fori_loop(0, _NSB // _NIN, quad_loop, 0)
    wait_out(_NSB - 2, 0)
    wait_out(_NSB - 1, 1)


def kernel(x, pos_table):
    mesh = plsc.VectorSubcoreMesh(core_axis_name="c", subcore_axis_name="s")
    run = functools.partial(
        pl.kernel,
        mesh=mesh,
        out_type=jax.ShapeDtypeStruct((_BATCH, _SEQ, _DIM), jnp.float32),
        scratch_types=[
            pltpu.VMEM((_NIN, _SB, _DIM), jnp.float32),
            pltpu.VMEM((_NIN, _BATCH, _SB, _DIM), jnp.float32),
            pltpu.VMEM((_NOUT, _BATCH, _SB, _DIM), jnp.float32),
            pltpu.SemaphoreType.DMA,
            pltpu.SemaphoreType.DMA,
            pltpu.SemaphoreType.DMA,
            pltpu.SemaphoreType.DMA,
            pltpu.SemaphoreType.DMA,
            pltpu.SemaphoreType.DMA,
        ],
    )(_sc_body)
    return run(x, pos_table)
